# baseline (speedup 1.0000x reference)
import functools
import jax
import jax.numpy as jnp
from jax import lax
from jax.experimental import pallas as pl
from jax.experimental.pallas import tpu as pltpu
from jax.experimental.pallas import tpu_sc as plsc

D_EMB = 64
NC = 2
NS = 16
NW = NC * NS
CB = 256
L = 16


def _body(nrows, nb, xT_hbm, table_hbm, outT_hbm,
          idx0, idx1, rows0, rows1, tb0, tb1,
          gsem0, gsem1, wsem0, wsem1, isem0, isem1):
    wid = lax.axis_index("s") * NC + lax.axis_index("c")
    bw = wid * (2 * CB)
    lane = lax.broadcasted_iota(jnp.int32, (L,), 0)

    def fire_idx(s, idx_v, sem):
        pltpu.async_copy(xT_hbm.at[s, pl.ds(bw, 2 * CB)], idx_v, sem)

    def wait_idx(idx_v, sem):
        pltpu.make_async_copy(xT_hbm.at[0, pl.ds(0, 2 * CB)], idx_v, sem).wait()

    def fire_gathers(h, idx_v, rows_v, sem):
        for j in range(2):
            pltpu.async_copy(
                table_hbm.at[idx_v.at[pl.ds(h * CB + j * 128, 128)]],
                rows_v.at[pl.ds(j * 128, 128)], sem)

    def wait_gathers(h, idx_v, rows_v, sem):
        for j in range(2):
            pltpu.make_async_copy(
                table_hbm.at[idx_v.at[pl.ds(h * CB + j * 128, 128)]],
                rows_v.at[pl.ds(j * 128, 128)], sem).wait()

    def transpose(rows_v, tb_v):
        def tr(r0, carry):
            row_idx = lane + r0 * L
            for d in range(D_EMB):
                col_idx = jnp.full((L,), d, jnp.int32)
                vals = plsc.load_gather(rows_v, [row_idx, col_idx])
                tb_v[d, pl.ds(r0 * L, L)] = vals
            return carry
        lax.fori_loop(0, CB // L, tr, 0)

    def fire_write(s, h, tb_v, sem):
        pltpu.async_copy(tb_v, outT_hbm.at[s, :, pl.ds(bw + h * CB, CB)], sem)

    def wait_write(tb_v, sem):
        pltpu.make_async_copy(tb_v, outT_hbm.at[0, :, pl.ds(0, CB)], sem).wait()

    pltpu.sync_copy(xT_hbm.at[0, pl.ds(bw, 2 * CB)], idx0)
    fire_gathers(0, idx0, rows0, gsem0)
    fire_gathers(1, idx0, rows1, gsem1)
    fire_idx(1, idx1, isem1)
    wait_gathers(0, idx0, rows0, gsem0)
    transpose(rows0, tb0)
    fire_write(0, 0, tb0, wsem0)
    wait_gathers(1, idx0, rows1, gsem1)
    transpose(rows1, tb1)
    fire_write(0, 1, tb1, wsem1)

    def body2(s, idxa, idxb, isa, isb):
        wait_idx(idxa, isa)
        fire_gathers(0, idxa, rows0, gsem0)
        fire_gathers(1, idxa, rows1, gsem1)
        fire_idx(jnp.minimum(s + 1, nrows - 1), idxb, isb)
        wait_gathers(0, idxa, rows0, gsem0)
        wait_write(tb0, wsem0)
        transpose(rows0, tb0)
        fire_write(s, 0, tb0, wsem0)
        wait_gathers(1, idxa, rows1, gsem1)
        wait_write(tb1, wsem1)
        transpose(rows1, tb1)
        fire_write(s, 1, tb1, wsem1)

    def body(o, carry):
        s = 2 * o - 1
        body2(s, idx1, idx0, isem1, isem0)
        body2(s + 1, idx0, idx1, isem0, isem1)
        return carry

    lax.fori_loop(1, nrows // 2, body, 0)
    body2(nrows - 1, idx1, idx0, isem1, isem0)

    wait_write(tb0, wsem0)
    wait_write(tb1, wsem1)
    wait_idx(idx0, isem0)


def kernel(x, table):
    B0, S = x.shape
    assert B0 % (NW * 2 * CB) == 0 and B0 // (NW * 2 * CB) == 1
    assert S % 2 == 0
    xT = jnp.transpose(x)

    mesh = plsc.VectorSubcoreMesh(core_axis_name="c", subcore_axis_name="s")
    run = pl.kernel(
        functools.partial(_body, S, B0),
        mesh=mesh,
        compiler_params=pltpu.CompilerParams(use_tc_tiling_on_sc=False, needs_layout_passes=False),
        out_type=jax.ShapeDtypeStruct((S, D_EMB, B0), jnp.float32),
        scratch_types=[
            pltpu.VMEM((2 * CB,), jnp.int32),
            pltpu.VMEM((2 * CB,), jnp.int32),
            pltpu.VMEM((CB, D_EMB), jnp.float32),
            pltpu.VMEM((CB, D_EMB), jnp.float32),
            pltpu.VMEM((D_EMB, CB), jnp.float32),
            pltpu.VMEM((D_EMB, CB), jnp.float32),
            pltpu.SemaphoreType.DMA,
            pltpu.SemaphoreType.DMA,
            pltpu.SemaphoreType.DMA,
            pltpu.SemaphoreType.DMA,
            pltpu.SemaphoreType.DMA,
            pltpu.SemaphoreType.DMA,
        ],
    )
    outT = run(xT, table)
    return jnp.transpose(outT, (2, 0, 1))


# speedup vs baseline: 1.5326x; 1.5326x over previous
import functools
import jax
import jax.numpy as jnp
from jax import lax
from jax.experimental import pallas as pl
from jax.experimental.pallas import tpu as pltpu
from jax.experimental.pallas import tpu_sc as plsc

D_EMB = 64
NC = 2
NS = 16
NW = NC * NS
CB = 256
L = 16


def _body(nrows, nb, xT_hbm, table_hbm, outT_hbm,
          idx0, idx1, rows0, rows1, tb0, tb1,
          gsem0, gsem1, wsem0, wsem1, isem0, isem1):
    wid = lax.axis_index("s") * NC + lax.axis_index("c")
    bw = wid * (2 * CB)
    lane = lax.broadcasted_iota(jnp.int32, (L,), 0)

    def fire_idx(s, idx_v, sem):
        pltpu.async_copy(xT_hbm.at[s, pl.ds(bw, 2 * CB)], idx_v, sem)

    def wait_idx(idx_v, sem):
        pltpu.make_async_copy(xT_hbm.at[0, pl.ds(0, 2 * CB)], idx_v, sem).wait()

    def fire_gathers(h, idx_v, rows_v, sem):
        for j in range(2):
            pltpu.async_copy(
                table_hbm.at[idx_v.at[pl.ds(h * CB + j * 128, 128)]],
                rows_v.at[pl.ds(j * 128, 128)], sem)

    def wait_gathers(h, idx_v, rows_v, sem):
        for j in range(2):
            pltpu.make_async_copy(
                table_hbm.at[idx_v.at[pl.ds(h * CB + j * 128, 128)]],
                rows_v.at[pl.ds(j * 128, 128)], sem).wait()

    row_idx = [lane + r0 * L for r0 in range(CB // L)]

    def transpose(rows_v, tb_v):
        @plsc.parallel_loop(0, D_EMB, 1, unroll=4)
        def _tr(d):
            col_idx = jnp.full((L,), d, jnp.int32)
            for r0 in range(CB // L):
                vals = plsc.load_gather(rows_v, [row_idx[r0], col_idx])
                tb_v[d, pl.ds(r0 * L, L)] = vals

    def fire_write(s, h, tb_v, sem):
        pltpu.async_copy(tb_v, outT_hbm.at[s, :, pl.ds(bw + h * CB, CB)], sem)

    def wait_write(tb_v, sem):
        pltpu.make_async_copy(tb_v, outT_hbm.at[0, :, pl.ds(0, CB)], sem).wait()

    pltpu.sync_copy(xT_hbm.at[0, pl.ds(bw, 2 * CB)], idx0)
    fire_gathers(0, idx0, rows0, gsem0)
    fire_gathers(1, idx0, rows1, gsem1)
    fire_idx(1, idx1, isem1)
    wait_gathers(0, idx0, rows0, gsem0)
    transpose(rows0, tb0)
    fire_write(0, 0, tb0, wsem0)
    wait_gathers(1, idx0, rows1, gsem1)
    transpose(rows1, tb1)
    fire_write(0, 1, tb1, wsem1)

    def body2(s, idxa, idxb, isa, isb):
        wait_idx(idxa, isa)
        fire_gathers(0, idxa, rows0, gsem0)
        fire_gathers(1, idxa, rows1, gsem1)
        fire_idx(jnp.minimum(s + 1, nrows - 1), idxb, isb)
        wait_gathers(0, idxa, rows0, gsem0)
        wait_write(tb0, wsem0)
        transpose(rows0, tb0)
        fire_write(s, 0, tb0, wsem0)
        wait_gathers(1, idxa, rows1, gsem1)
        wait_write(tb1, wsem1)
        transpose(rows1, tb1)
        fire_write(s, 1, tb1, wsem1)

    def body(o, carry):
        s = 2 * o - 1
        body2(s, idx1, idx0, isem1, isem0)
        body2(s + 1, idx0, idx1, isem0, isem1)
        return carry

    lax.fori_loop(1, nrows // 2, body, 0)
    body2(nrows - 1, idx1, idx0, isem1, isem0)

    wait_write(tb0, wsem0)
    wait_write(tb1, wsem1)
    wait_idx(idx0, isem0)


def kernel(x, table):
    B0, S = x.shape
    assert B0 % (NW * 2 * CB) == 0 and B0 // (NW * 2 * CB) == 1
    assert S % 2 == 0
    xT = jnp.transpose(x)

    mesh = plsc.VectorSubcoreMesh(core_axis_name="c", subcore_axis_name="s")
    run = pl.kernel(
        functools.partial(_body, S, B0),
        mesh=mesh,
        compiler_params=pltpu.CompilerParams(use_tc_tiling_on_sc=False, needs_layout_passes=False),
        out_type=jax.ShapeDtypeStruct((S, D_EMB, B0), jnp.float32),
        scratch_types=[
            pltpu.VMEM((2 * CB,), jnp.int32),
            pltpu.VMEM((2 * CB,), jnp.int32),
            pltpu.VMEM((CB, D_EMB), jnp.float32),
            pltpu.VMEM((CB, D_EMB), jnp.float32),
            pltpu.VMEM((D_EMB, CB), jnp.float32),
            pltpu.VMEM((D_EMB, CB), jnp.float32),
            pltpu.SemaphoreType.DMA,
            pltpu.SemaphoreType.DMA,
            pltpu.SemaphoreType.DMA,
            pltpu.SemaphoreType.DMA,
            pltpu.SemaphoreType.DMA,
            pltpu.SemaphoreType.DMA,
        ],
    )
    outT = run(xT, table)
    return jnp.transpose(outT, (2, 0, 1))


# speedup vs baseline: 2.1631x; 1.4114x over previous
import functools
import jax
import jax.numpy as jnp
from jax import lax
from jax.experimental import pallas as pl
from jax.experimental.pallas import tpu as pltpu
from jax.experimental.pallas import tpu_sc as plsc

D_EMB = 64
NC = 2
NS = 16
NW = NC * NS
XR = 4
SPLITS = ((0, 104), (104, 96))


def _body(nchunks, ncols, x_hbm, table_hbm, out_hbm,
          idx0, idx1, rows0, rows1,
          gsem0, gsem1, wsem0, wsem1, isem0, isem1):
    wid = lax.axis_index("s") * NC + lax.axis_index("c")
    row_base = wid * (nchunks * XR)
    last = nchunks - 1

    def fire_gathers(idx_v, rows_v, sem):
        for r in range(XR):
            for off, ln in SPLITS:
                pltpu.async_copy(table_hbm.at[idx_v.at[r, pl.ds(off, ln)]],
                                 rows_v.at[r, pl.ds(off, ln)], sem)

    def wait_gathers(idx_v, rows_v, sem):
        for r in range(XR):
            for off, ln in SPLITS:
                pltpu.make_async_copy(table_hbm.at[idx_v.at[r, pl.ds(off, ln)]],
                                      rows_v.at[r, pl.ds(off, ln)], sem).wait()

    def fire_idx(g, idx_v, sem):
        pltpu.async_copy(x_hbm.at[pl.ds(row_base + g * XR, XR)], idx_v, sem)

    def wait_idx(idx_v, sem):
        pltpu.make_async_copy(x_hbm.at[pl.ds(0, XR)], idx_v, sem).wait()

    def fire_write(g, rows_v, sem):
        pltpu.async_copy(rows_v, out_hbm.at[pl.ds(row_base + g * XR, XR)], sem)

    def wait_write(rows_v, sem):
        pltpu.make_async_copy(rows_v, out_hbm.at[pl.ds(0, XR)], sem).wait()

    pltpu.sync_copy(x_hbm.at[pl.ds(row_base, XR)], idx0)
    fire_gathers(idx0, rows0, gsem0)
    fire_idx(1, idx1, isem1)
    wait_idx(idx1, isem1)
    fire_gathers(idx1, rows1, gsem1)
    wait_gathers(idx0, rows0, gsem0)
    fire_write(0, rows0, wsem0)
    fire_idx(2, idx0, isem0)

    def body(o, carry):
        g = 2 * o
        wait_write(rows0, wsem0)
        wait_idx(idx0, isem0)
        fire_gathers(idx0, rows0, gsem0)
        wait_gathers(idx1, rows1, gsem1)
        fire_write(g - 1, rows1, wsem1)
        fire_idx(jnp.minimum(g + 1, last), idx1, isem1)
        wait_write(rows1, wsem1)
        wait_idx(idx1, isem1)
        fire_gathers(idx1, rows1, gsem1)
        wait_gathers(idx0, rows0, gsem0)
        fire_write(g, rows0, wsem0)
        fire_idx(jnp.minimum(g + 2, last), idx0, isem0)
        return carry

    lax.fori_loop(1, nchunks // 2, body, 0)

    wait_write(rows0, wsem0)
    wait_gathers(idx1, rows1, gsem1)
    fire_write(last, rows1, wsem1)
    wait_idx(idx0, isem0)
    wait_write(rows1, wsem1)


def kernel(x, table):
    B0, S = x.shape
    assert S == 200 and B0 % (NW * XR) == 0
    nchunks = B0 // (NW * XR)
    assert nchunks >= 2 and nchunks % 2 == 0

    mesh = plsc.VectorSubcoreMesh(core_axis_name="c", subcore_axis_name="s")
    run = pl.kernel(
        functools.partial(_body, nchunks, S),
        mesh=mesh,
        compiler_params=pltpu.CompilerParams(use_tc_tiling_on_sc=False),
        out_type=jax.ShapeDtypeStruct((B0, S, D_EMB), jnp.float32),
        scratch_types=[
            pltpu.VMEM((XR, S), jnp.int32),
            pltpu.VMEM((XR, S), jnp.int32),
            pltpu.VMEM((XR, S, D_EMB), jnp.float32),
            pltpu.VMEM((XR, S, D_EMB), jnp.float32),
            pltpu.SemaphoreType.DMA,
            pltpu.SemaphoreType.DMA,
            pltpu.SemaphoreType.DMA,
            pltpu.SemaphoreType.DMA,
            pltpu.SemaphoreType.DMA,
            pltpu.SemaphoreType.DMA,
        ],
    )
    return run(x, table)


# speedup vs baseline: 3.5625x; 1.6469x over previous
import functools
import jax
import jax.numpy as jnp
from jax import lax
from jax.experimental import pallas as pl
from jax.experimental.pallas import tpu as pltpu
from jax.experimental.pallas import tpu_sc as plsc

D_EMB = 64
NC = 2
NS = 16
NW = NC * NS
XR = 4
SPLITS = ((0, 104), (104, 96))


def _body(nchunks, ncols, x_hbm, table_hbm, out_hbm,
          idx0, idx1, rows0, rows1,
          gsem0, gsem1, wsem0, wsem1, isem0, isem1):
    wid = lax.axis_index("s") * NC + lax.axis_index("c")
    row_base = wid * (nchunks * XR)
    last = nchunks - 1

    def fire_gathers(idx_v, rows_v, sem):
        for r in range(XR):
            for off, ln in SPLITS:
                pltpu.async_copy(table_hbm.at[idx_v.at[r, pl.ds(off, ln)]],
                                 rows_v.at[r, pl.ds(off, ln)], sem)

    def wait_gathers(idx_v, rows_v, sem):
        for r in range(XR):
            for off, ln in SPLITS:
                pltpu.make_async_copy(table_hbm.at[idx_v.at[r, pl.ds(off, ln)]],
                                      rows_v.at[r, pl.ds(off, ln)], sem).wait()

    def fire_idx(g, idx_v, sem):
        pltpu.async_copy(x_hbm.at[pl.ds(row_base + g * XR, XR)], idx_v, sem)

    def wait_idx(idx_v, sem):
        pltpu.make_async_copy(x_hbm.at[pl.ds(0, XR)], idx_v, sem).wait()

    def fire_write(g, rows_v, sem):
        pltpu.async_copy(rows_v,
                         out_hbm.at[pl.ds(row_base + g * XR, XR), :, pl.ds(0, 64)],
                         sem)

    def wait_write(rows_v, sem):
        pltpu.make_async_copy(rows_v,
                              out_hbm.at[pl.ds(0, XR), :, pl.ds(0, 64)],
                              sem).wait()

    pltpu.sync_copy(x_hbm.at[pl.ds(row_base, XR)], idx0)
    fire_gathers(idx0, rows0, gsem0)
    fire_idx(1, idx1, isem1)
    wait_idx(idx1, isem1)
    fire_gathers(idx1, rows1, gsem1)
    wait_gathers(idx0, rows0, gsem0)
    fire_write(0, rows0, wsem0)
    fire_idx(2, idx0, isem0)

    def body(o, carry):
        g = 2 * o
        wait_write(rows0, wsem0)
        wait_idx(idx0, isem0)
        fire_gathers(idx0, rows0, gsem0)
        wait_gathers(idx1, rows1, gsem1)
        fire_write(g - 1, rows1, wsem1)
        fire_idx(jnp.minimum(g + 1, last), idx1, isem1)
        wait_write(rows1, wsem1)
        wait_idx(idx1, isem1)
        fire_gathers(idx1, rows1, gsem1)
        wait_gathers(idx0, rows0, gsem0)
        fire_write(g, rows0, wsem0)
        fire_idx(jnp.minimum(g + 2, last), idx0, isem0)
        return carry

    lax.fori_loop(1, nchunks // 2, body, 0)

    wait_write(rows0, wsem0)
    wait_gathers(idx1, rows1, gsem1)
    fire_write(last, rows1, wsem1)
    wait_idx(idx0, isem0)
    wait_write(rows1, wsem1)


def kernel(x, table):
    B0, S = x.shape
    assert S == 200 and B0 % (NW * XR) == 0
    nchunks = B0 // (NW * XR)
    assert nchunks >= 2 and nchunks % 2 == 0

    mesh = plsc.VectorSubcoreMesh(core_axis_name="c", subcore_axis_name="s")
    run = pl.kernel(
        functools.partial(_body, nchunks, S),
        mesh=mesh,
        compiler_params=pltpu.CompilerParams(use_tc_tiling_on_sc=False),
        out_type=jax.ShapeDtypeStruct((B0, S, 2 * D_EMB), jnp.float32),
        scratch_types=[
            pltpu.VMEM((XR, S), jnp.int32),
            pltpu.VMEM((XR, S), jnp.int32),
            pltpu.VMEM((XR, S, D_EMB), jnp.float32),
            pltpu.VMEM((XR, S, D_EMB), jnp.float32),
            pltpu.SemaphoreType.DMA,
            pltpu.SemaphoreType.DMA,
            pltpu.SemaphoreType.DMA,
            pltpu.SemaphoreType.DMA,
            pltpu.SemaphoreType.DMA,
            pltpu.SemaphoreType.DMA,
        ],
    )
    t2 = jnp.reshape(jnp.reshape(table, (500000, 128)), (1000000, 64))
    out_p = run(x, t2)
    return out_p[:, :, :64]
